# Initial kernel scaffold; baseline (speedup 1.0000x reference)
import jax
import jax.numpy as jnp
from jax.experimental import pallas as pl


def kernel(embedding_ids, embedding_table):
    raise NotImplementedError("write your pallas kernel here")


# speedup vs baseline: 5.3485x; 5.3485x over previous
import functools

import jax
import jax.numpy as jnp
from jax import lax
from jax.experimental import pallas as pl
from jax.experimental.pallas import tpu as pltpu
from jax.experimental.pallas import tpu_sc as plsc


def _sc_geometry():
    try:
        info = plsc.get_sparse_core_info()
        return info.num_cores, info.num_subcores
    except Exception:
        return 2, 16


def kernel(embedding_ids, embedding_table):
    B, H = embedding_ids.shape
    V, D = embedding_table.shape
    N = B * H
    NC, NS = _sc_geometry()
    NW = NC * NS

    CH = 128
    per_w = N // NW
    n_chunks = per_w // CH
    assert per_w * NW == N and n_chunks * CH == per_w and n_chunks % 2 == 0

    idx = embedding_ids.reshape(NW, n_chunks, CH).astype(jnp.int32)
    n_pairs = n_chunks // 2

    mesh = plsc.VectorSubcoreMesh(core_axis_name="c", subcore_axis_name="s")

    @functools.partial(
        pl.kernel,
        out_type=jax.ShapeDtypeStruct((N, D), jnp.float32),
        mesh=mesh,
        scratch_types=[
            pltpu.VMEM((n_chunks, CH), jnp.int32),
            pltpu.VMEM((CH, D), jnp.float32),
            pltpu.VMEM((CH, D), jnp.float32),
            pltpu.SemaphoreType.DMA,
            pltpu.SemaphoreType.DMA,
        ],
    )
    def run(idx_hbm, tbl_hbm, out_hbm, idx_v, rows0, rows1, sem0, sem1):
        wid = lax.axis_index("s") * NC + lax.axis_index("c")
        base = wid * per_w
        pltpu.sync_copy(idx_hbm.at[wid], idx_v)
        pltpu.async_copy(tbl_hbm.at[idx_v.at[0]], rows0, sem0)

        def pair(p, carry):
            j0 = 2 * p
            j1 = j0 + 1
            pltpu.async_copy(tbl_hbm.at[idx_v.at[j1]], rows1, sem1)
            pltpu.make_async_copy(tbl_hbm.at[idx_v.at[0]], rows0, sem0).wait()
            pltpu.sync_copy(rows0, out_hbm.at[pl.ds(base + j0 * CH, CH)])

            @pl.when(p + 1 < n_pairs)
            def _():
                pltpu.async_copy(tbl_hbm.at[idx_v.at[j0 + 2]], rows0, sem0)

            pltpu.make_async_copy(tbl_hbm.at[idx_v.at[0]], rows1, sem1).wait()
            pltpu.sync_copy(rows1, out_hbm.at[pl.ds(base + j1 * CH, CH)])
            return carry

        lax.fori_loop(0, n_pairs, pair, 0)

    out = run(idx, embedding_table)
    return out.reshape(B, H * D)


# speedup vs baseline: 5.4205x; 1.0135x over previous
import functools

import jax
import jax.numpy as jnp
from jax import lax
from jax.experimental import pallas as pl
from jax.experimental.pallas import tpu as pltpu
from jax.experimental.pallas import tpu_sc as plsc


def _sc_geometry():
    try:
        info = plsc.get_sparse_core_info()
        return info.num_cores, info.num_subcores
    except Exception:
        return 2, 16


def kernel(embedding_ids, embedding_table):
    B, H = embedding_ids.shape
    V, D = embedding_table.shape
    N = B * H
    NC, NS = _sc_geometry()
    NW = NC * NS

    CH = 128
    RING = 5
    LOOKAHEAD = 3
    per_w = N // NW
    n_chunks = per_w // CH
    assert per_w * NW == N and n_chunks * CH == per_w
    assert n_chunks % RING == 0 and n_chunks >= RING

    idx = embedding_ids.reshape(NW, n_chunks, CH).astype(jnp.int32)

    mesh = plsc.VectorSubcoreMesh(core_axis_name="c", subcore_axis_name="s")

    @functools.partial(
        pl.kernel,
        out_type=jax.ShapeDtypeStruct((N, D), jnp.float32),
        mesh=mesh,
        scratch_types=[
            pltpu.VMEM((n_chunks, CH), jnp.int32),
            pltpu.VMEM((RING, CH, D), jnp.float32),
        ]
        + [pltpu.SemaphoreType.DMA] * (2 * RING),
    )
    def run(idx_hbm, tbl_hbm, out_hbm, idx_v, rows, *sems):
        gsem = sems[:RING]
        wsem = sems[RING:]
        wid = lax.axis_index("s") * NC + lax.axis_index("c")
        base = wid * per_w
        pltpu.sync_copy(idx_hbm.at[wid], idx_v)

        def gather(j, b):
            pltpu.async_copy(tbl_hbm.at[idx_v.at[j]], rows.at[b], gsem[b])

        def wait_gather(b):
            pltpu.make_async_copy(
                tbl_hbm.at[idx_v.at[0]], rows.at[b], gsem[b]
            ).wait()

        def write(j, b):
            pltpu.async_copy(
                rows.at[b], out_hbm.at[pl.ds(base + j * CH, CH)], wsem[b]
            )

        def wait_write(b):
            pltpu.make_async_copy(
                rows.at[b], out_hbm.at[pl.ds(base, CH)], wsem[b]
            ).wait()

        for b in range(LOOKAHEAD):
            gather(b, b)

        def body(p, carry):
            for b in range(RING):
                j = RING * p + b
                bp = (b + LOOKAHEAD) % RING
                if b < RING - LOOKAHEAD:

                    @pl.when(p > 0)
                    def _(bp=bp):
                        wait_write(bp)

                else:
                    wait_write(bp)
                max_j = RING * (n_chunks // RING - 1) + b
                if max_j + LOOKAHEAD < n_chunks:
                    gather(j + LOOKAHEAD, bp)
                else:

                    @pl.when(j + LOOKAHEAD < n_chunks)
                    def _(j=j, bp=bp):
                        gather(j + LOOKAHEAD, bp)

                wait_gather(b)
                write(j, b)
            return carry

        lax.fori_loop(0, n_chunks // RING, body, 0)

        for j in range(n_chunks - (RING - LOOKAHEAD), n_chunks):
            wait_write(j % RING)

    out = run(idx, embedding_table)
    return out.reshape(B, H * D)


# speedup vs baseline: 11.2825x; 2.0815x over previous
import functools

import jax
import jax.numpy as jnp
from jax import lax
from jax.experimental import pallas as pl
from jax.experimental.pallas import tpu as pltpu
from jax.experimental.pallas import tpu_sc as plsc


def _sc_geometry():
    try:
        info = plsc.get_sparse_core_info()
        return info.num_cores, info.num_subcores
    except Exception:
        return 2, 16


def kernel(embedding_ids, embedding_table):
    B, H = embedding_ids.shape
    V, D = embedding_table.shape
    NC, NS = _sc_geometry()
    NW = NC * NS

    rows_w = B // NW
    n_chunks = H
    RING = 5
    LOOKAHEAD = 3
    assert rows_w * NW == B
    assert n_chunks % RING == 0 and n_chunks >= RING

    idx = (
        embedding_ids.astype(jnp.int32)
        .reshape(NW, rows_w, H)
        .transpose(0, 2, 1)
    )

    mesh = plsc.VectorSubcoreMesh(core_axis_name="c", subcore_axis_name="s")

    @functools.partial(
        pl.kernel,
        out_type=jax.ShapeDtypeStruct((B, H * D), jnp.float32),
        mesh=mesh,
        scratch_types=[
            pltpu.VMEM((n_chunks, rows_w), jnp.int32),
            pltpu.VMEM((RING, rows_w, D), jnp.float32),
        ]
        + [pltpu.SemaphoreType.DMA] * (2 * RING),
    )
    def run(idx_hbm, tbl_hbm, out_hbm, idx_v, rows, *sems):
        gsem = sems[:RING]
        wsem = sems[RING:]
        wid = lax.axis_index("s") * NC + lax.axis_index("c")
        row0 = wid * rows_w
        pltpu.sync_copy(idx_hbm.at[wid], idx_v)

        def gather(j, b):
            pltpu.async_copy(tbl_hbm.at[idx_v.at[j]], rows.at[b], gsem[b])

        def wait_gather(b):
            pltpu.make_async_copy(
                tbl_hbm.at[idx_v.at[0]], rows.at[b], gsem[b]
            ).wait()

        def write(j, b):
            pltpu.async_copy(
                rows.at[b],
                out_hbm.at[pl.ds(row0, rows_w), pl.ds(j * D, D)],
                wsem[b],
            )

        def wait_write(b):
            pltpu.make_async_copy(
                rows.at[b],
                out_hbm.at[pl.ds(row0, rows_w), pl.ds(0, D)],
                wsem[b],
            ).wait()

        for b in range(LOOKAHEAD):
            gather(b, b)

        def body(p, carry):
            for b in range(RING):
                j = RING * p + b
                bp = (b + LOOKAHEAD) % RING
                if b < RING - LOOKAHEAD:

                    @pl.when(p > 0)
                    def _(bp=bp):
                        wait_write(bp)

                else:
                    wait_write(bp)
                max_j = RING * (n_chunks // RING - 1) + b
                if max_j + LOOKAHEAD < n_chunks:
                    gather(j + LOOKAHEAD, bp)
                else:

                    @pl.when(j + LOOKAHEAD < n_chunks)
                    def _(j=j, bp=bp):
                        gather(j + LOOKAHEAD, bp)

                wait_gather(b)
                write(j, b)
            return carry

        lax.fori_loop(0, n_chunks // RING, body, 0)

        for j in range(n_chunks - (RING - LOOKAHEAD), n_chunks):
            wait_write(j % RING)

    return run(idx, embedding_table)
